# baseline (speedup 1.0000x reference)
import jax
import jax.numpy as jnp
from jax.experimental import pallas as pl
from jax.experimental.pallas import tpu as pltpu

SEQ, B, I, H = 512, 64, 1024, 1024
TBLK = 16
NT = SEQ // TBLK


def _gru_block(len_ref, hinit_ref, x_ref, wih_ref, whh_ref, bih_ref,
               bhh_ref, out_ref, h_scr):
    i = pl.program_id(0)

    @pl.when(i == 0)
    def _init():
        h_scr[...] = jnp.broadcast_to(hinit_ref[...], (B, H))

    x = x_ref[...].reshape(TBLK * B, I).astype(jnp.bfloat16)
    gi = jnp.dot(x, wih_ref[...], preferred_element_type=jnp.float32)
    gi = gi + bih_ref[...]

    length = len_ref[...]
    bhh = bhh_ref[...]
    h = h_scr[...]
    for t in range(TBLK):
        gt = gi[t * B:(t + 1) * B, :]
        hb = h.astype(jnp.bfloat16)
        gh = jnp.dot(jnp.concatenate([hb, hb, hb, hb], axis=0), whh_ref[...],
                     preferred_element_type=jnp.float32)[:B]
        gh = gh + bhh
        r = jax.nn.sigmoid(gt[:, :H] + gh[:, :H])
        z = jax.nn.sigmoid(gt[:, H:2 * H] + gh[:, H:2 * H])
        n = jnp.tanh(gt[:, 2 * H:] + r * gh[:, 2 * H:])
        h_new = (1.0 - z) * n + z * h
        m = (i * TBLK + t) < length
        h = jnp.where(m, h_new, h)
    h_scr[...] = h

    @pl.when(i == NT - 1)
    def _fin():
        out_ref[...] = jnp.where(length > 0, h, 0.0)


def kernel(incoming, length, w_ih, w_hh, b_ih, b_hh, h_init):
    len2 = length.astype(jnp.int32).reshape(B, 1)
    wih_t = w_ih.T.astype(jnp.bfloat16)
    whh_t = w_hh.T.astype(jnp.bfloat16)
    bih2 = b_ih.reshape(1, 3 * H)
    bhh2 = b_hh.reshape(1, 3 * H)
    hinit2 = h_init.reshape(1, H)

    in_specs = [
        pl.BlockSpec((B, 1), lambda i: (0, 0)),
        pl.BlockSpec((1, H), lambda i: (0, 0)),
        pl.BlockSpec((TBLK, B, I), lambda i: (i, 0, 0)),
        pl.BlockSpec((I, 3 * H), lambda i: (0, 0)),
        pl.BlockSpec((H, 3 * H), lambda i: (0, 0)),
        pl.BlockSpec((1, 3 * H), lambda i: (0, 0)),
        pl.BlockSpec((1, 3 * H), lambda i: (0, 0)),
    ]

    return pl.pallas_call(
        _gru_block,
        grid=(NT,),
        in_specs=in_specs,
        out_specs=pl.BlockSpec((B, H), lambda i: (0, 0)),
        out_shape=jax.ShapeDtypeStruct((B, H), jnp.float32),
        scratch_shapes=[pltpu.VMEM((B, H), jnp.float32)],
        compiler_params=pltpu.CompilerParams(
            dimension_semantics=("arbitrary",),
        ),
    )(len2, hinit2, incoming, wih_t, whh_t, bih2, bhh2)


# speedup vs baseline: 1.5056x; 1.5056x over previous
import jax
import jax.numpy as jnp
from jax.experimental import pallas as pl
from jax.experimental.pallas import tpu as pltpu

SEQ, B, I, H = 512, 64, 1024, 1024
TBLK = 16
NT = SEQ // TBLK


def _gru_block(len_ref, hinit_ref, x_ref, wih_ref, whh_ref, bih_ref,
               bhh_ref, out_ref, h_scr):
    i = pl.program_id(0)

    @pl.when(i == 0)
    def _init():
        h_scr[...] = jnp.broadcast_to(hinit_ref[...], (B, H))

    x = x_ref[...].reshape(TBLK * B, I).astype(jnp.bfloat16)
    gi = jnp.dot(x, wih_ref[...], preferred_element_type=jnp.float32)
    gi = gi + bih_ref[...]

    length = len_ref[...]
    bhh = bhh_ref[...]
    h = h_scr[...]
    for t in range(TBLK):
        gt = gi[t * B:(t + 1) * B, :]
        hb = h.astype(jnp.bfloat16)
        gh = jnp.dot(jnp.concatenate([hb, hb], axis=0), whh_ref[...],
                     preferred_element_type=jnp.float32)[:B]
        gh = gh + bhh
        r = jax.nn.sigmoid(gt[:, :H] + gh[:, :H])
        z = jax.nn.sigmoid(gt[:, H:2 * H] + gh[:, H:2 * H])
        n = jnp.tanh(gt[:, 2 * H:] + r * gh[:, 2 * H:])
        h_new = (1.0 - z) * n + z * h
        m = (i * TBLK + t) < length
        h = jnp.where(m, h_new, h)
    h_scr[...] = h

    @pl.when(i == NT - 1)
    def _fin():
        out_ref[...] = jnp.where(length > 0, h, 0.0)


def kernel(incoming, length, w_ih, w_hh, b_ih, b_hh, h_init):
    len2 = length.astype(jnp.int32).reshape(B, 1)
    wih_t = w_ih.T.astype(jnp.bfloat16)
    whh_t = w_hh.T.astype(jnp.bfloat16)
    bih2 = b_ih.reshape(1, 3 * H)
    bhh2 = b_hh.reshape(1, 3 * H)
    hinit2 = h_init.reshape(1, H)

    in_specs = [
        pl.BlockSpec((B, 1), lambda i: (0, 0)),
        pl.BlockSpec((1, H), lambda i: (0, 0)),
        pl.BlockSpec((TBLK, B, I), lambda i: (i, 0, 0)),
        pl.BlockSpec((I, 3 * H), lambda i: (0, 0)),
        pl.BlockSpec((H, 3 * H), lambda i: (0, 0)),
        pl.BlockSpec((1, 3 * H), lambda i: (0, 0)),
        pl.BlockSpec((1, 3 * H), lambda i: (0, 0)),
    ]

    return pl.pallas_call(
        _gru_block,
        grid=(NT,),
        in_specs=in_specs,
        out_specs=pl.BlockSpec((B, H), lambda i: (0, 0)),
        out_shape=jax.ShapeDtypeStruct((B, H), jnp.float32),
        scratch_shapes=[pltpu.VMEM((B, H), jnp.float32)],
        compiler_params=pltpu.CompilerParams(
            dimension_semantics=("arbitrary",),
        ),
    )(len2, hinit2, incoming, wih_t, whh_t, bih2, bhh2)
